# baseline (speedup 1.0000x reference)
import jax
import jax.numpy as jnp
from jax import lax
from jax.experimental import pallas as pl
from jax.experimental.pallas import tpu as pltpu, tpu_sc as plsc

N = 10000
E = 320000
D_IN = 128
D_H = 256
D_OUT = 128
BN_EPS = 1e-5

NC = 2
NS = 16
NW = NC * NS
EW = E // NW
CH = 64
NCH = 157
EWP = NCH * CH
NPAD = 10240
DEGW = 16
RPS = NPAD // NS


def _deg_body(dst_hbm, ones_hbm, zero_hbm, out_hbm, acc,
              dst0, ss0, dst1, ss1, dst2, ss2, ones_v):
  c = lax.axis_index("c")
  s = lax.axis_index("s")
  wid = s * NC + c
  base = wid * EWP
  pltpu.sync_copy(ones_hbm, ones_v)
  pltpu.sync_copy(zero_hbm, acc.at[pl.ds(s * RPS, RPS)])
  plsc.subcore_barrier()

  B = ((dst0, ss0), (dst1, ss1), (dst2, ss2))

  def stage(ci, b):
    dbuf, _ = B[b]
    pltpu.sync_copy(dst_hbm.at[pl.ds(pl.multiple_of(base + ci * CH, 8), CH)],
                    dbuf)

  def fire_scat(b):
    dbuf, ssem = B[b]
    pltpu.async_copy(ones_v, acc.at[dbuf], ssem, add=True)

  def wait_scat(b):
    dbuf, ssem = B[b]
    pltpu.make_async_copy(ones_v, acc.at[dbuf], ssem).wait()

  stage(0, 0)
  stage(1, 1)
  fire_scat(0)
  stage(2, 2)

  def step(k, carry):
    c0 = 3 * k + 1
    for j, (b, b2) in enumerate(((1, 0), (2, 1), (0, 2))):
      ci = c0 + j
      fire_scat(b)
      wait_scat(b2)

      @pl.when(ci + 2 < NCH)
      def _():
        stage(ci + 2, b2)
    return carry

  lax.fori_loop(0, (NCH - 1) // 3, step, 0)
  k3 = 3 * ((NCH - 1) // 3)
  for ci in range(k3 + 1, NCH):
    fire_scat(ci % 3)
  for cj in range(k3, NCH):
    wait_scat(cj % 3)
  plsc.subcore_barrier()
  pltpu.sync_copy(acc.at[pl.ds(s * RPS, RPS)],
                  out_hbm.at[c, pl.ds(s * RPS, RPS)])


def _agg_body(xs_hbm, src_hbm, dst_hbm, zero_hbm, out_hbm, acc,
              src0, dst0, rows0, gs0, ss0,
              src1, dst1, rows1, gs1, ss1,
              src2, dst2, rows2, gs2, ss2):
  c = lax.axis_index("c")
  s = lax.axis_index("s")
  wid = s * NC + c
  base = wid * EWP
  pltpu.sync_copy(zero_hbm, acc.at[pl.ds(s * RPS, RPS)])
  plsc.subcore_barrier()

  B = ((src0, dst0, rows0, gs0, ss0),
       (src1, dst1, rows1, gs1, ss1),
       (src2, dst2, rows2, gs2, ss2))

  def stage(ci, b):
    sbuf, dbuf, rbuf, gsem, _ = B[b]
    off = pl.multiple_of(base + ci * CH, 8)
    pltpu.sync_copy(src_hbm.at[pl.ds(off, CH)], sbuf)
    pltpu.sync_copy(dst_hbm.at[pl.ds(off, CH)], dbuf)
    pltpu.async_copy(xs_hbm.at[sbuf], rbuf, gsem)

  def fire_scat(b):
    sbuf, dbuf, rbuf, gsem, ssem = B[b]
    pltpu.make_async_copy(xs_hbm.at[sbuf], rbuf, gsem).wait()
    pltpu.async_copy(rbuf, acc.at[dbuf], ssem, add=True)

  def wait_scat(b):
    sbuf, dbuf, rbuf, _, ssem = B[b]
    pltpu.make_async_copy(rbuf, acc.at[dbuf], ssem).wait()

  stage(0, 0)
  stage(1, 1)
  fire_scat(0)
  stage(2, 2)

  def step(k, carry):
    c0 = 3 * k + 1
    for j, (b, b2) in enumerate(((1, 0), (2, 1), (0, 2))):
      ci = c0 + j
      fire_scat(b)
      wait_scat(b2)

      @pl.when(ci + 2 < NCH)
      def _():
        stage(ci + 2, b2)
    return carry

  lax.fori_loop(0, (NCH - 1) // 3, step, 0)
  k3 = 3 * ((NCH - 1) // 3)
  for ci in range(k3 + 1, NCH):
    fire_scat(ci % 3)
  for cj in range(k3, NCH):
    wait_scat(cj % 3)
  plsc.subcore_barrier()
  pltpu.sync_copy(acc.at[pl.ds(s * RPS, RPS)],
                  out_hbm.at[c, pl.ds(s * RPS, RPS)])


_sc_kernels_cache = {}


def _sc_kernels():
  if "k" not in _sc_kernels_cache:
    mesh = plsc.VectorSubcoreMesh(core_axis_name="c", subcore_axis_name="s",
                                  num_cores=NC, num_subcores=NS)
    deg = pl.kernel(
        _deg_body,
        out_type=jax.ShapeDtypeStruct((NC, NPAD, 128), jnp.float32),
        mesh=mesh,
        scratch_types=[
            pltpu.VMEM_SHARED((NPAD, 128), jnp.float32),
            pltpu.VMEM((CH,), jnp.int32),
            pltpu.SemaphoreType.DMA,
            pltpu.VMEM((CH,), jnp.int32),
            pltpu.SemaphoreType.DMA,
            pltpu.VMEM((CH,), jnp.int32),
            pltpu.SemaphoreType.DMA,
            pltpu.VMEM((CH, 128), jnp.float32),
        ],
    )
    agg = pl.kernel(
        _agg_body,
        out_type=jax.ShapeDtypeStruct((NC, NPAD, D_IN), jnp.float32),
        mesh=mesh,
        scratch_types=[
            pltpu.VMEM_SHARED((NPAD, D_IN), jnp.float32),
            pltpu.VMEM((CH,), jnp.int32),
            pltpu.VMEM((CH,), jnp.int32),
            pltpu.VMEM((CH, D_IN), jnp.float32),
            pltpu.SemaphoreType.DMA,
            pltpu.SemaphoreType.DMA,
            pltpu.VMEM((CH,), jnp.int32),
            pltpu.VMEM((CH,), jnp.int32),
            pltpu.VMEM((CH, D_IN), jnp.float32),
            pltpu.SemaphoreType.DMA,
            pltpu.SemaphoreType.DMA,
            pltpu.VMEM((CH,), jnp.int32),
            pltpu.VMEM((CH,), jnp.int32),
            pltpu.VMEM((CH, D_IN), jnp.float32),
            pltpu.SemaphoreType.DMA,
            pltpu.SemaphoreType.DMA,
        ],
    )
    _sc_kernels_cache["k"] = (deg, agg)
  return _sc_kernels_cache["k"]


def _tc1_body(degp_ref, x_ref, dis_ref, xs1_ref):
  deg = degp_ref[0, 0:N, 0:1] + degp_ref[1, 0:N, 0:1] + 1.0
  dis = lax.rsqrt(deg)
  dis_ref[...] = dis
  xs1_ref[...] = x_ref[...] * dis


def _tc2_body(p_ref, xs1_ref, dis_ref, w0t_ref, b0_ref, g0_ref, be0_ref,
              w1t_ref, xs2_ref):
  dis = dis_ref[...]
  z1 = dis * (p_ref[0, 0:N, :] + p_ref[1, 0:N, :] + xs1_ref[...])
  h1 = jnp.dot(z1, w0t_ref[...],
               preferred_element_type=jnp.float32) + b0_ref[...]
  mean = jnp.mean(h1, axis=0, keepdims=True)
  var = jnp.mean((h1 - mean) ** 2, axis=0, keepdims=True)
  h = (h1 - mean) * lax.rsqrt(var + BN_EPS) * g0_ref[...] + be0_ref[...]
  h = jnp.maximum(h, 0.0)
  h2 = jnp.dot(h, w1t_ref[...], preferred_element_type=jnp.float32)
  xs2_ref[...] = h2 * dis


def _tc3_body(q_ref, xs2_ref, dis_ref, b1_ref, out_ref):
  out_ref[...] = dis_ref[...] * (q_ref[0, 0:N, :] + q_ref[1, 0:N, :]
                                 + xs2_ref[...]) + b1_ref[...]


def _tc1(degp, x):
  return pl.pallas_call(
      _tc1_body,
      out_shape=[jax.ShapeDtypeStruct((N, 1), jnp.float32),
                 jax.ShapeDtypeStruct((N, D_IN), jnp.float32)],
  )(degp, x)


def _tc2(p, xs1, dis, w0t, b0, g0, be0, w1t):
  return pl.pallas_call(
      _tc2_body,
      out_shape=jax.ShapeDtypeStruct((N, D_OUT), jnp.float32),
  )(p, xs1, dis, w0t, b0, g0, be0, w1t)


def _tc3(q, xs2, dis, b1):
  return pl.pallas_call(
      _tc3_body,
      out_shape=jax.ShapeDtypeStruct((N, D_OUT), jnp.float32),
  )(q, xs2, dis, b1)


def kernel(x, edge_index, W0, b0, gamma0, beta0, W1, b1):
  _deg, _agg = _sc_kernels()
  src = edge_index[0].astype(jnp.int32).reshape(NW, EW)
  dst = edge_index[1].astype(jnp.int32).reshape(NW, EW)
  src = jnp.concatenate([src, jnp.zeros((NW, EWP - EW), jnp.int32)],
                        axis=1).reshape(NW * EWP)
  dst = jnp.concatenate([dst, jnp.full((NW, EWP - EW), N, jnp.int32)],
                        axis=1).reshape(NW * EWP)
  zeros_feat = jnp.zeros((RPS, D_IN), jnp.float32)
  ones_feat = jnp.ones((CH, 128), jnp.float32)

  degp = _deg(dst, ones_feat, zeros_feat)
  dis, xs1 = _tc1(degp, x)
  p = _agg(xs1, src, dst, zeros_feat)
  xs2 = _tc2(p, xs1, dis, W0.T, b0[None], gamma0[None], beta0[None], W1.T)
  q = _agg(xs2, src, dst, zeros_feat)
  return _tc3(q, xs2, dis, b1[None])


# speedup vs baseline: 1.3312x; 1.3312x over previous
import jax
import jax.numpy as jnp
from jax import lax
from jax.experimental import pallas as pl
from jax.experimental.pallas import tpu as pltpu, tpu_sc as plsc

N = 10000
E = 320000
D_IN = 128
D_H = 256
D_OUT = 128
BN_EPS = 1e-5

NC = 2
NS = 16
NW = NC * NS
EW = E // NW
CH = 80
NCH = 125
EWP = NCH * CH
NPAD = 10240
DEGW = 16
RPS = NPAD // NS


def _deg_body(dst_hbm, ones_hbm, zero_hbm, out_hbm, acc,
              dst0, ss0, dst1, ss1, dst2, ss2, ones_v):
  c = lax.axis_index("c")
  s = lax.axis_index("s")
  wid = s * NC + c
  base = wid * EWP
  pltpu.sync_copy(ones_hbm, ones_v)
  pltpu.sync_copy(zero_hbm, acc.at[pl.ds(s * RPS, RPS)])
  plsc.subcore_barrier()

  B = ((dst0, ss0), (dst1, ss1), (dst2, ss2))

  def stage(ci, b):
    dbuf, _ = B[b]
    pltpu.sync_copy(dst_hbm.at[pl.ds(pl.multiple_of(base + ci * CH, 8), CH)],
                    dbuf)

  def fire_scat(b):
    dbuf, ssem = B[b]
    pltpu.async_copy(ones_v, acc.at[dbuf], ssem, add=True)

  def wait_scat(b):
    dbuf, ssem = B[b]
    pltpu.make_async_copy(ones_v, acc.at[dbuf], ssem).wait()

  stage(0, 0)
  stage(1, 1)
  fire_scat(0)
  stage(2, 2)

  def step(k, carry):
    c0 = 3 * k + 1
    for j, (b, b2) in enumerate(((1, 0), (2, 1), (0, 2))):
      ci = c0 + j
      fire_scat(b)
      wait_scat(b2)

      @pl.when(ci + 2 < NCH)
      def _():
        stage(ci + 2, b2)
    return carry

  lax.fori_loop(0, (NCH - 1) // 3, step, 0)
  k3 = 3 * ((NCH - 1) // 3)
  for ci in range(k3 + 1, NCH):
    fire_scat(ci % 3)
  for cj in range(k3, NCH):
    wait_scat(cj % 3)
  plsc.subcore_barrier()
  pltpu.sync_copy(acc.at[pl.ds(s * RPS, RPS)],
                  out_hbm.at[c, pl.ds(s * RPS, RPS)])


def _agg_body(xs_hbm, src_hbm, dst_hbm, zero_hbm, out_hbm, acc,
              src0, dst0, rows0, gs0, ss0,
              src1, dst1, rows1, gs1, ss1,
              src2, dst2, rows2, gs2, ss2):
  c = lax.axis_index("c")
  s = lax.axis_index("s")
  wid = s * NC + c
  base = wid * EWP
  pltpu.sync_copy(zero_hbm, acc.at[pl.ds(s * RPS, RPS)])
  plsc.subcore_barrier()

  B = ((src0, dst0, rows0, gs0, ss0),
       (src1, dst1, rows1, gs1, ss1),
       (src2, dst2, rows2, gs2, ss2))

  def stage(ci, b):
    sbuf, dbuf, rbuf, gsem, _ = B[b]
    off = pl.multiple_of(base + ci * CH, 8)
    pltpu.sync_copy(src_hbm.at[pl.ds(off, CH)], sbuf)
    pltpu.sync_copy(dst_hbm.at[pl.ds(off, CH)], dbuf)
    pltpu.async_copy(xs_hbm.at[sbuf], rbuf, gsem)

  def fire_scat(b):
    sbuf, dbuf, rbuf, gsem, ssem = B[b]
    pltpu.make_async_copy(xs_hbm.at[sbuf], rbuf, gsem).wait()
    pltpu.async_copy(rbuf, acc.at[dbuf], ssem, add=True)

  def wait_scat(b):
    sbuf, dbuf, rbuf, _, ssem = B[b]
    pltpu.make_async_copy(rbuf, acc.at[dbuf], ssem).wait()

  stage(0, 0)
  stage(1, 1)
  fire_scat(0)
  stage(2, 2)

  def step(k, carry):
    c0 = 3 * k + 1
    for j, (b, b2) in enumerate(((1, 0), (2, 1), (0, 2))):
      ci = c0 + j
      fire_scat(b)
      wait_scat(b2)

      @pl.when(ci + 2 < NCH)
      def _():
        stage(ci + 2, b2)
    return carry

  lax.fori_loop(0, (NCH - 1) // 3, step, 0)
  k3 = 3 * ((NCH - 1) // 3)
  for ci in range(k3 + 1, NCH):
    fire_scat(ci % 3)
  for cj in range(k3, NCH):
    wait_scat(cj % 3)
  plsc.subcore_barrier()
  pltpu.sync_copy(acc.at[pl.ds(s * RPS, RPS)],
                  out_hbm.at[c, pl.ds(s * RPS, RPS)])


_sc_kernels_cache = {}


def _sc_kernels():
  if "k" not in _sc_kernels_cache:
    mesh = plsc.VectorSubcoreMesh(core_axis_name="c", subcore_axis_name="s",
                                  num_cores=NC, num_subcores=NS)
    deg = pl.kernel(
        _deg_body,
        out_type=jax.ShapeDtypeStruct((NC, NPAD, 128), jnp.float32),
        mesh=mesh,
        scratch_types=[
            pltpu.VMEM_SHARED((NPAD, 128), jnp.float32),
            pltpu.VMEM((CH,), jnp.int32),
            pltpu.SemaphoreType.DMA,
            pltpu.VMEM((CH,), jnp.int32),
            pltpu.SemaphoreType.DMA,
            pltpu.VMEM((CH,), jnp.int32),
            pltpu.SemaphoreType.DMA,
            pltpu.VMEM((CH, 128), jnp.float32),
        ],
    )
    agg = pl.kernel(
        _agg_body,
        out_type=jax.ShapeDtypeStruct((NC, NPAD, D_IN), jnp.float32),
        mesh=mesh,
        scratch_types=[
            pltpu.VMEM_SHARED((NPAD, D_IN), jnp.float32),
            pltpu.VMEM((CH,), jnp.int32),
            pltpu.VMEM((CH,), jnp.int32),
            pltpu.VMEM((CH, D_IN), jnp.float32),
            pltpu.SemaphoreType.DMA,
            pltpu.SemaphoreType.DMA,
            pltpu.VMEM((CH,), jnp.int32),
            pltpu.VMEM((CH,), jnp.int32),
            pltpu.VMEM((CH, D_IN), jnp.float32),
            pltpu.SemaphoreType.DMA,
            pltpu.SemaphoreType.DMA,
            pltpu.VMEM((CH,), jnp.int32),
            pltpu.VMEM((CH,), jnp.int32),
            pltpu.VMEM((CH, D_IN), jnp.float32),
            pltpu.SemaphoreType.DMA,
            pltpu.SemaphoreType.DMA,
        ],
    )
    _sc_kernels_cache["k"] = (deg, agg)
  return _sc_kernels_cache["k"]


def _tc1_body(degp_ref, x_ref, dis_ref, xs1_ref):
  deg = degp_ref[0, 0:N, 0:1] + degp_ref[1, 0:N, 0:1] + 1.0
  dis = lax.rsqrt(deg)
  dis_ref[...] = dis
  xs1_ref[...] = x_ref[...] * dis


def _tc2_body(p_ref, xs1_ref, dis_ref, w0t_ref, b0_ref, g0_ref, be0_ref,
              w1t_ref, xs2_ref):
  dis = dis_ref[...]
  z1 = dis * (p_ref[0, 0:N, :] + p_ref[1, 0:N, :] + xs1_ref[...])
  h1 = jnp.dot(z1, w0t_ref[...],
               preferred_element_type=jnp.float32) + b0_ref[...]
  mean = jnp.mean(h1, axis=0, keepdims=True)
  var = jnp.mean((h1 - mean) ** 2, axis=0, keepdims=True)
  h = (h1 - mean) * lax.rsqrt(var + BN_EPS) * g0_ref[...] + be0_ref[...]
  h = jnp.maximum(h, 0.0)
  h2 = jnp.dot(h, w1t_ref[...], preferred_element_type=jnp.float32)
  xs2_ref[...] = h2 * dis


def _tc3_body(q_ref, xs2_ref, dis_ref, b1_ref, out_ref):
  out_ref[...] = dis_ref[...] * (q_ref[0, 0:N, :] + q_ref[1, 0:N, :]
                                 + xs2_ref[...]) + b1_ref[...]


def _tc1(degp, x):
  return pl.pallas_call(
      _tc1_body,
      out_shape=[jax.ShapeDtypeStruct((N, 1), jnp.float32),
                 jax.ShapeDtypeStruct((N, D_IN), jnp.float32)],
  )(degp, x)


def _tc2(p, xs1, dis, w0t, b0, g0, be0, w1t):
  return pl.pallas_call(
      _tc2_body,
      out_shape=jax.ShapeDtypeStruct((N, D_OUT), jnp.float32),
  )(p, xs1, dis, w0t, b0, g0, be0, w1t)


def _tc3(q, xs2, dis, b1):
  return pl.pallas_call(
      _tc3_body,
      out_shape=jax.ShapeDtypeStruct((N, D_OUT), jnp.float32),
  )(q, xs2, dis, b1)


def kernel(x, edge_index, W0, b0, gamma0, beta0, W1, b1):
  _deg, _agg = _sc_kernels()
  src = edge_index[0].astype(jnp.int32)
  dst = edge_index[1].astype(jnp.int32)
  zeros_feat = jnp.zeros((RPS, D_IN), jnp.float32)
  ones_feat = jnp.ones((CH, 128), jnp.float32)

  degp = _deg(dst, ones_feat, zeros_feat)
  dis, xs1 = _tc1(degp, x)
  p = _agg(xs1, src, dst, zeros_feat)
  xs2 = _tc2(p, xs1, dis, W0.T, b0[None], gamma0[None], beta0[None], W1.T)
  q = _agg(xs2, src, dst, zeros_feat)
  return _tc3(q, xs2, dis, b1[None])


# speedup vs baseline: 1.3347x; 1.0026x over previous
import jax
import jax.numpy as jnp
from jax import lax
from jax.experimental import pallas as pl
from jax.experimental.pallas import tpu as pltpu, tpu_sc as plsc

N = 10000
E = 320000
D_IN = 128
D_H = 256
D_OUT = 128
BN_EPS = 1e-5

NC = 2
NS = 16
NW = NC * NS
EW = E // NW
CH = 80
NCH = 125
EWP = NCH * CH
NPAD = 10240
RPS = NPAD // NS


def _deg_body(dst_hbm, ones_hbm, zero_hbm, out_hbm, acc,
              dst0, ss0, dst1, ss1, dst2, ss2, dst3, ss3, ones_v):
  c = lax.axis_index("c")
  s = lax.axis_index("s")
  wid = s * NC + c
  base = wid * EWP
  pltpu.sync_copy(ones_hbm, ones_v)
  pltpu.sync_copy(zero_hbm, acc.at[pl.ds(s * RPS, RPS)])
  plsc.subcore_barrier()

  B = ((dst0, ss0), (dst1, ss1), (dst2, ss2), (dst3, ss3))

  def stage(ci, b):
    dbuf, _ = B[b]
    pltpu.sync_copy(dst_hbm.at[pl.ds(pl.multiple_of(base + ci * CH, 8), CH)],
                    dbuf)

  def fire_scat(b):
    dbuf, ssem = B[b]
    pltpu.async_copy(ones_v, acc.at[dbuf], ssem, add=True)

  def wait_scat(b):
    dbuf, ssem = B[b]
    pltpu.make_async_copy(ones_v, acc.at[dbuf], ssem).wait()

  stage(0, 0)
  stage(1, 1)
  fire_scat(0)
  stage(2, 2)
  fire_scat(1)
  stage(3, 3)

  def step(k, carry):
    c0 = 4 * k + 2
    for j in range(4):
      ci = c0 + j
      b = (2 + j) % 4
      b2 = j
      fire_scat(b)
      wait_scat(b2)

      @pl.when(ci + 2 < NCH)
      def _():
        stage(ci + 2, b2)
    return carry

  nk = (NCH - 2) // 4
  lax.fori_loop(0, nk, step, 0)
  for ci in range(4 * nk + 2, NCH):
    fire_scat(ci % 4)
    wait_scat((ci + 2) % 4)
    if ci + 2 < NCH:
      stage(ci + 2, (ci + 2) % 4)
  wait_scat((NCH - 2) % 4)
  wait_scat((NCH - 1) % 4)
  plsc.subcore_barrier()
  pltpu.sync_copy(acc.at[pl.ds(s * RPS, RPS)],
                  out_hbm.at[c, pl.ds(s * RPS, RPS)])


def _agg_body(xs_hbm, src_hbm, dst_hbm, zero_hbm, out_hbm, acc,
              src0, dst0, rows0, gs0, ss0,
              src1, dst1, rows1, gs1, ss1,
              src2, dst2, rows2, gs2, ss2):
  c = lax.axis_index("c")
  s = lax.axis_index("s")
  wid = s * NC + c
  base = wid * EWP
  pltpu.sync_copy(zero_hbm, acc.at[pl.ds(s * RPS, RPS)])
  plsc.subcore_barrier()

  B = ((src0, dst0, rows0, gs0, ss0),
       (src1, dst1, rows1, gs1, ss1),
       (src2, dst2, rows2, gs2, ss2))

  def stage(ci, b):
    sbuf, dbuf, rbuf, gsem, _ = B[b]
    off = pl.multiple_of(base + ci * CH, 8)
    pltpu.sync_copy(src_hbm.at[pl.ds(off, CH)], sbuf)
    pltpu.sync_copy(dst_hbm.at[pl.ds(off, CH)], dbuf)
    pltpu.async_copy(xs_hbm.at[sbuf], rbuf, gsem)

  def fire_scat(b):
    sbuf, dbuf, rbuf, gsem, ssem = B[b]
    pltpu.make_async_copy(xs_hbm.at[sbuf], rbuf, gsem).wait()
    pltpu.async_copy(rbuf, acc.at[dbuf], ssem, add=True)

  def wait_scat(b):
    sbuf, dbuf, rbuf, _, ssem = B[b]
    pltpu.make_async_copy(rbuf, acc.at[dbuf], ssem).wait()

  stage(0, 0)
  stage(1, 1)
  fire_scat(0)
  stage(2, 2)

  def step(k, carry):
    c0 = 3 * k + 1
    for j, (b, b2) in enumerate(((1, 0), (2, 1), (0, 2))):
      ci = c0 + j
      fire_scat(b)
      wait_scat(b2)

      @pl.when(ci + 2 < NCH)
      def _():
        stage(ci + 2, b2)
    return carry

  lax.fori_loop(0, (NCH - 1) // 3, step, 0)
  k3 = 3 * ((NCH - 1) // 3)
  for ci in range(k3 + 1, NCH):
    fire_scat(ci % 3)
  for cj in range(k3, NCH):
    wait_scat(cj % 3)
  plsc.subcore_barrier()
  pltpu.sync_copy(acc.at[pl.ds(s * RPS, RPS)],
                  out_hbm.at[c, pl.ds(s * RPS, RPS)])


_sc_kernels_cache = {}


def _sc_kernels():
  if "k" not in _sc_kernels_cache:
    mesh = plsc.VectorSubcoreMesh(core_axis_name="c", subcore_axis_name="s",
                                  num_cores=NC, num_subcores=NS)
    deg = pl.kernel(
        _deg_body,
        out_type=jax.ShapeDtypeStruct((NC, NPAD, 128), jnp.float32),
        mesh=mesh,
        scratch_types=[
            pltpu.VMEM_SHARED((NPAD, 128), jnp.float32),
            pltpu.VMEM((CH,), jnp.int32),
            pltpu.SemaphoreType.DMA,
            pltpu.VMEM((CH,), jnp.int32),
            pltpu.SemaphoreType.DMA,
            pltpu.VMEM((CH,), jnp.int32),
            pltpu.SemaphoreType.DMA,
            pltpu.VMEM((CH,), jnp.int32),
            pltpu.SemaphoreType.DMA,
            pltpu.VMEM((CH, 128), jnp.float32),
        ],
    )
    agg = pl.kernel(
        _agg_body,
        out_type=jax.ShapeDtypeStruct((NC, NPAD, D_IN), jnp.float32),
        mesh=mesh,
        scratch_types=[
            pltpu.VMEM_SHARED((NPAD, D_IN), jnp.float32),
            pltpu.VMEM((CH,), jnp.int32),
            pltpu.VMEM((CH,), jnp.int32),
            pltpu.VMEM((CH, D_IN), jnp.float32),
            pltpu.SemaphoreType.DMA,
            pltpu.SemaphoreType.DMA,
            pltpu.VMEM((CH,), jnp.int32),
            pltpu.VMEM((CH,), jnp.int32),
            pltpu.VMEM((CH, D_IN), jnp.float32),
            pltpu.SemaphoreType.DMA,
            pltpu.SemaphoreType.DMA,
            pltpu.VMEM((CH,), jnp.int32),
            pltpu.VMEM((CH,), jnp.int32),
            pltpu.VMEM((CH, D_IN), jnp.float32),
            pltpu.SemaphoreType.DMA,
            pltpu.SemaphoreType.DMA,
        ],
    )
    _sc_kernels_cache["k"] = (deg, agg)
  return _sc_kernels_cache["k"]


def _tc1_body(degp_ref, x_ref, dis_ref, xs1_ref):
  deg = degp_ref[0, 0:N, 0:1] + degp_ref[1, 0:N, 0:1] + 1.0
  dis = lax.rsqrt(deg)
  dis_ref[...] = dis
  xs1_ref[...] = x_ref[...] * dis


def _tc2_body(p_ref, xs1_ref, dis_ref, w0t_ref, b0_ref, g0_ref, be0_ref,
              w1t_ref, xs2_ref):
  dis = dis_ref[...]
  z1 = dis * (p_ref[0, 0:N, :] + p_ref[1, 0:N, :] + xs1_ref[...])
  h1 = jnp.dot(z1, w0t_ref[...],
               preferred_element_type=jnp.float32) + b0_ref[...]
  mean = jnp.mean(h1, axis=0, keepdims=True)
  var = jnp.mean((h1 - mean) ** 2, axis=0, keepdims=True)
  h = (h1 - mean) * lax.rsqrt(var + BN_EPS) * g0_ref[...] + be0_ref[...]
  h = jnp.maximum(h, 0.0)
  h2 = jnp.dot(h, w1t_ref[...], preferred_element_type=jnp.float32)
  xs2_ref[...] = h2 * dis


def _tc3_body(q_ref, xs2_ref, dis_ref, b1_ref, out_ref):
  out_ref[...] = dis_ref[...] * (q_ref[0, 0:N, :] + q_ref[1, 0:N, :]
                                 + xs2_ref[...]) + b1_ref[...]


def _tc1(degp, x):
  return pl.pallas_call(
      _tc1_body,
      out_shape=[jax.ShapeDtypeStruct((N, 1), jnp.float32),
                 jax.ShapeDtypeStruct((N, D_IN), jnp.float32)],
  )(degp, x)


def _tc2(p, xs1, dis, w0t, b0, g0, be0, w1t):
  return pl.pallas_call(
      _tc2_body,
      out_shape=jax.ShapeDtypeStruct((N, D_OUT), jnp.float32),
  )(p, xs1, dis, w0t, b0, g0, be0, w1t)


def _tc3(q, xs2, dis, b1):
  return pl.pallas_call(
      _tc3_body,
      out_shape=jax.ShapeDtypeStruct((N, D_OUT), jnp.float32),
  )(q, xs2, dis, b1)


def kernel(x, edge_index, W0, b0, gamma0, beta0, W1, b1):
  _deg, _agg = _sc_kernels()
  src = edge_index[0].astype(jnp.int32)
  dst = edge_index[1].astype(jnp.int32)
  zeros_feat = jnp.zeros((RPS, D_IN), jnp.float32)
  ones_feat = jnp.ones((CH, 128), jnp.float32)

  degp = _deg(dst, ones_feat, zeros_feat)
  dis, xs1 = _tc1(degp, x)
  p = _agg(xs1, src, dst, zeros_feat)
  xs2 = _tc2(p, xs1, dis, W0.T, b0[None], gamma0[None], beta0[None], W1.T)
  q = _agg(xs2, src, dst, zeros_feat)
  return _tc3(q, xs2, dis, b1[None])
